# baseline (speedup 1.0000x reference)
import jax
import jax.numpy as jnp
from jax.experimental import pallas as pl
from jax.experimental.pallas import tpu as pltpu

_D = 1024
_E = 64
_TOPK = 2
_HID = 128
_S = 6


def _ln(x, g, b):
    m = jnp.mean(x, axis=-1, keepdims=True)
    v = jnp.mean((x - m) ** 2, axis=-1, keepdims=True)
    return (x - m) / jnp.sqrt(v + 1e-5) * g + b


def _stage1_kernel(x_ref, w1_ref, b1_ref, g1_ref, bb1_ref, w2_ref, b2_ref,
                   g2_ref, bb2_ref, lng_ref, lnb_ref, wg_ref, bg_ref,
                   h_out_ref, gs_out_ref):
    x = x_ref[...]
    h = _ln(jnp.dot(x, w1_ref[...], preferred_element_type=jnp.float32)
            + b1_ref[...], g1_ref[...], bb1_ref[...])
    h = jnp.maximum(h, 0.0)
    h = _ln(jnp.dot(h, w2_ref[...], preferred_element_type=jnp.float32)
            + b2_ref[...], g2_ref[...], bb2_ref[...])
    h = h + x
    h = jnp.maximum(_ln(h, lng_ref[...], lnb_ref[...]), 0.0)
    h_out_ref[...] = h
    gs_out_ref[...] = (jnp.dot(h, wg_ref[...], preferred_element_type=jnp.float32)
                       + bg_ref[...])


def _moe_body(key_ref, tw_ref, we_refs, be_ref, h_ref, ws_ref, bs_ref,
              acc_ref, steps):
    s = pl.program_id(0)

    @pl.when(s == 0)
    def _init():
        h = h_ref[...]
        acc_ref[...] = (jnp.dot(h, ws_ref[...], preferred_element_type=jnp.float32)
                        + bs_ref[...] + h)

    for c, we_ref in enumerate(we_refs):
        k = key_ref[c * steps + s]
        e = k // 128
        slot = k - e * 128
        valid = slot < 2 * 32
        tok = jnp.where(valid, slot // 2, 0)
        w = jnp.where(valid, tw_ref[jnp.where(valid, slot, 0)], 0.0)
        row = h_ref[pl.ds(tok, 1), :]
        y = (jnp.dot(row, we_ref[0], preferred_element_type=jnp.float32)
             + be_ref[pl.ds(e, 1), :])
        acc_ref[pl.ds(tok, 1), :] += w * y


def _moe_gate_kernel(key_ref, tw_ref, *refs):
    we_refs = refs[:_S]
    (be_ref, h_ref, ws_ref, bs_ref, wg_ref, bg_ref,
     h_out_ref, gs_out_ref, acc_ref) = refs[_S:]
    steps = pl.num_programs(0)
    _moe_body(key_ref, tw_ref, we_refs, be_ref, h_ref, ws_ref, bs_ref,
              acc_ref, steps)

    @pl.when(pl.program_id(0) == steps - 1)
    def _fin():
        o = jnp.maximum(acc_ref[...], 0.0)
        h_out_ref[...] = o
        gs_out_ref[...] = (jnp.dot(o, wg_ref[...], preferred_element_type=jnp.float32)
                           + bg_ref[...])


def _moe_tail_kernel(key_ref, tw_ref, *refs):
    we_refs = refs[:_S]
    (be_ref, h_ref, ws_ref, bs_ref, w1_ref, b1_ref, g1_ref, bb1_ref,
     w2_ref, b2_ref, g2_ref, bb2_ref, y_out_ref, acc_ref) = refs[_S:]
    steps = pl.num_programs(0)
    _moe_body(key_ref, tw_ref, we_refs, be_ref, h_ref, ws_ref, bs_ref,
              acc_ref, steps)

    @pl.when(pl.program_id(0) == steps - 1)
    def _fin():
        o = jnp.maximum(acc_ref[...], 0.0)
        u = _ln(jnp.dot(o, w1_ref[...], preferred_element_type=jnp.float32)
                + b1_ref[...], g1_ref[...], bb1_ref[...])
        u = jnp.maximum(u, 0.0)
        u = _ln(jnp.dot(u, w2_ref[...], preferred_element_type=jnp.float32)
                + b2_ref[...], g2_ref[...], bb2_ref[...])
        y_out_ref[...] = u + o


def _row(v):
    return v.reshape(1, -1)


def _routing(gate_scores):
    n = gate_scores.shape[0]
    a = n * _TOPK
    topv, topi = jax.lax.top_k(gate_scores, _TOPK)
    topw = jax.nn.softmax(topv, axis=-1)
    keys = (topi.astype(jnp.int32).reshape(-1) * 128
            + jnp.arange(a, dtype=jnp.int32))
    keys = jnp.sort(keys)
    pad = (-a) % _S
    if pad:
        padk = ((keys[-1] // 128) * 128 + 2 * n
                + jnp.arange(pad, dtype=jnp.int32))
        keys = jnp.concatenate([keys, padk])
    return keys, topw.reshape(-1)


def _we_spec(d, c, steps):
    return pl.BlockSpec(
        (1, d, d),
        lambda i, key_ref, tw_ref: (key_ref[c * steps + i] // 128, 0, 0))


def _moe_call(kern, h, keys, tw, we, be, ws, bs, tail_ops, n_out_extra):
    n, d = h.shape
    steps = keys.shape[0] // _S
    const2 = lambda i, *_: (0, 0)
    in_specs = [_we_spec(d, c, steps) for c in range(_S)] + [
        pl.BlockSpec(be.shape, const2),
        pl.BlockSpec((n, d), const2),
        pl.BlockSpec((d, d), const2),
        pl.BlockSpec((1, d), const2),
    ] + [pl.BlockSpec(t.shape, const2) for t in tail_ops]
    if n_out_extra is None:
        out_shape = jax.ShapeDtypeStruct((n, d), jnp.float32)
        out_specs = pl.BlockSpec((n, d), const2)
    else:
        out_shape = (jax.ShapeDtypeStruct((n, d), jnp.float32),
                     jax.ShapeDtypeStruct((n, n_out_extra), jnp.float32))
        out_specs = (pl.BlockSpec((n, d), const2),
                     pl.BlockSpec((n, n_out_extra), const2))
    grid_spec = pltpu.PrefetchScalarGridSpec(
        num_scalar_prefetch=2,
        grid=(steps,),
        in_specs=in_specs,
        out_specs=out_specs,
        scratch_shapes=[pltpu.VMEM((n, d), jnp.float32)],
    )
    return pl.pallas_call(kern, grid_spec=grid_spec, out_shape=out_shape)(
        keys, tw, *([we] * _S), be, h, ws, bs, *tail_ops)


def kernel(x, params):
    n = x.shape[0] * x.shape[1]
    xf = x.reshape(n, x.shape[-1]).astype(jnp.float32)
    p0 = params['rl0']
    pf = params['rlf']
    m0 = params['moe0']
    m1 = params['moe1']

    h0, gs0 = pl.pallas_call(
        _stage1_kernel,
        out_shape=(jax.ShapeDtypeStruct((n, _D), jnp.float32),
                   jax.ShapeDtypeStruct((n, _E), jnp.float32)),
    )(xf, p0['W1'], _row(p0['b1']), _row(p0['g1']), _row(p0['bb1']),
      p0['W2'], _row(p0['b2']), _row(p0['g2']), _row(p0['bb2']),
      _row(params['ln0_g']), _row(params['ln0_b']),
      m0['Wg'], _row(m0['bg'] + m0['gate_bias']))

    keys0, tw0 = _routing(gs0)
    h1, gs1 = _moe_call(
        _moe_gate_kernel, h0, keys0, tw0, m0['We'], m0['be'],
        m0['Ws'], _row(m0['bs']),
        (m1['Wg'], _row(m1['bg'] + m1['gate_bias'])), _E)

    keys1, tw1 = _routing(gs1)
    y = _moe_call(
        _moe_tail_kernel, h1, keys1, tw1, m1['We'], m1['be'],
        m1['Ws'], _row(m1['bs']),
        (pf['W1'], _row(pf['b1']), _row(pf['g1']), _row(pf['bb1']),
         pf['W2'], _row(pf['b2']), _row(pf['g2']), _row(pf['bb2'])), None)

    return y.reshape(x.shape[:-1] + (y.shape[-1],))


# speedup vs baseline: 1.1019x; 1.1019x over previous
import jax
import jax.numpy as jnp
import numpy as np
from jax.experimental import pallas as pl
from jax.experimental.pallas import tpu as pltpu

_D = 1024
_E = 64
_TOPK = 2
_HID = 128
_S = 6
_N = 32
_A = _N * _TOPK
_PAD = (-_A) % _S


def _bitonic_consts(m):
    perms, masks = [], []
    k = 2
    while k <= m:
        j = k // 2
        while j >= 1:
            p = np.zeros((m, m), np.float32)
            for i in range(m):
                p[i ^ j, i] = 1.0
            mask = np.array(
                [[1.0 if (((i & j) == 0) == ((i & k) == 0)) else 0.0
                  for i in range(m)]], np.float32)
            perms.append(p)
            masks.append(mask)
            j //= 2
        k *= 2
    return np.stack(perms), np.stack(masks)


_PERMS, _MASKS = _bitonic_consts(_A)
_NSTAGES = _PERMS.shape[0]


def _ln(x, g, b):
    m = jnp.mean(x, axis=-1, keepdims=True)
    v = jnp.mean((x - m) ** 2, axis=-1, keepdims=True)
    return (x - m) / jnp.sqrt(v + 1e-5) * g + b


def _route(gs, perms_ref, masks_ref):
    gt = jnp.transpose(gs)
    ii = jax.lax.broadcasted_iota(jnp.int32, gt.shape, 0).astype(jnp.float32)
    m1 = jnp.max(gt, axis=0, keepdims=True)
    a1 = jnp.min(jnp.where(gt == m1, ii, float(_E)), axis=0, keepdims=True)
    g2 = jnp.where(ii == a1, -1e30, gt)
    m2 = jnp.max(g2, axis=0, keepdims=True)
    a2 = jnp.min(jnp.where(g2 == m2, ii, float(_E)), axis=0, keepdims=True)
    w1 = 1.0 / (1.0 + jnp.exp(m2 - m1))
    lane = jax.lax.broadcasted_iota(jnp.int32, (1, _N), 1).astype(jnp.float32)
    keys = jnp.concatenate(
        [a1 * 128.0 + lane, a2 * 128.0 + float(_N) + lane], axis=1)
    for st in range(_NSTAGES):
        kp = jnp.dot(keys, perms_ref[st], preferred_element_type=jnp.float32)
        keys = jnp.where(masks_ref[st] != 0.0,
                         jnp.minimum(keys, kp), jnp.maximum(keys, kp))
    tw = jnp.concatenate([w1, 1.0 - w1], axis=1)
    if _PAD:
        top = jnp.max(keys, axis=1, keepdims=True)
        pe = jnp.floor(top / 128.0) * 128.0 + float(_A)
        padk = pe + jax.lax.broadcasted_iota(jnp.int32, (1, _PAD), 1).astype(jnp.float32)
        keys = jnp.concatenate([keys, padk], axis=1)
    return keys, tw


def _stage1_kernel(x_ref, w1_ref, b1_ref, g1_ref, bb1_ref, w2_ref, b2_ref,
                   g2_ref, bb2_ref, lng_ref, lnb_ref, wg_ref, bg_ref,
                   perms_ref, masks_ref,
                   h_out_ref, keys_out_ref, tw_out_ref):
    x = x_ref[...]
    h = _ln(jnp.dot(x, w1_ref[...], preferred_element_type=jnp.float32)
            + b1_ref[...], g1_ref[...], bb1_ref[...])
    h = jnp.maximum(h, 0.0)
    h = _ln(jnp.dot(h, w2_ref[...], preferred_element_type=jnp.float32)
            + b2_ref[...], g2_ref[...], bb2_ref[...])
    h = h + x
    h = jnp.maximum(_ln(h, lng_ref[...], lnb_ref[...]), 0.0)
    h_out_ref[...] = h
    gs = (jnp.dot(h, wg_ref[...], preferred_element_type=jnp.float32)
          + bg_ref[...])
    keys, tw = _route(gs, perms_ref, masks_ref)
    keys_out_ref[...] = keys.astype(jnp.int32)
    tw_out_ref[...] = tw


def _moe_body(key_ref, tw_ref, we_refs, be_ref, h_ref, ws_ref, bs_ref,
              acc_ref, steps):
    s = pl.program_id(0)

    @pl.when(s == 0)
    def _init():
        h = h_ref[...]
        acc_ref[...] = (jnp.dot(h, ws_ref[...], preferred_element_type=jnp.float32)
                        + bs_ref[...] + h)

    for c, we_ref in enumerate(we_refs):
        k = key_ref[0, c * steps + s]
        e = k // 128
        slot = k - e * 128
        valid = slot < _A
        tok = jnp.where(valid, jnp.where(slot < _N, slot, slot - _N), 0)
        w = jnp.where(valid, tw_ref[0, jnp.where(valid, slot, 0)], 0.0)
        row = h_ref[pl.ds(tok, 1), :]
        y = (jnp.dot(row, we_ref[0], preferred_element_type=jnp.float32)
             + be_ref[pl.ds(e, 1), :])
        acc_ref[pl.ds(tok, 1), :] += w * y


def _moe_gate_kernel(key_ref, tw_ref, *refs):
    we_refs = refs[:_S]
    (be_ref, h_ref, ws_ref, bs_ref, wg_ref, bg_ref, perms_ref, masks_ref,
     h_out_ref, keys_out_ref, tw_out_ref, acc_ref) = refs[_S:]
    steps = pl.num_programs(0)
    _moe_body(key_ref, tw_ref, we_refs, be_ref, h_ref, ws_ref, bs_ref,
              acc_ref, steps)

    @pl.when(pl.program_id(0) == steps - 1)
    def _fin():
        o = jnp.maximum(acc_ref[...], 0.0)
        h_out_ref[...] = o
        gs = (jnp.dot(o, wg_ref[...], preferred_element_type=jnp.float32)
              + bg_ref[...])
        keys, tw = _route(gs, perms_ref, masks_ref)
        keys_out_ref[...] = keys.astype(jnp.int32)
        tw_out_ref[...] = tw


def _moe_tail_kernel(key_ref, tw_ref, *refs):
    we_refs = refs[:_S]
    (be_ref, h_ref, ws_ref, bs_ref, w1_ref, b1_ref, g1_ref, bb1_ref,
     w2_ref, b2_ref, g2_ref, bb2_ref, y_out_ref, acc_ref) = refs[_S:]
    steps = pl.num_programs(0)
    _moe_body(key_ref, tw_ref, we_refs, be_ref, h_ref, ws_ref, bs_ref,
              acc_ref, steps)

    @pl.when(pl.program_id(0) == steps - 1)
    def _fin():
        o = jnp.maximum(acc_ref[...], 0.0)
        u = _ln(jnp.dot(o, w1_ref[...], preferred_element_type=jnp.float32)
                + b1_ref[...], g1_ref[...], bb1_ref[...])
        u = jnp.maximum(u, 0.0)
        u = _ln(jnp.dot(u, w2_ref[...], preferred_element_type=jnp.float32)
                + b2_ref[...], g2_ref[...], bb2_ref[...])
        y_out_ref[...] = u + o


def _row(v):
    return v.reshape(1, -1)


def _we_spec(d, c, steps):
    return pl.BlockSpec(
        (1, d, d),
        lambda i, key_ref, tw_ref: (key_ref[0, c * steps + i] // 128, 0, 0))


def _moe_call(kern, h, keys, tw, we, be, ws, bs, tail_ops, routed_out):
    n, d = h.shape
    steps = keys.shape[1] // _S
    const2 = lambda i, *_: (0, 0)
    in_specs = [_we_spec(d, c, steps) for c in range(_S)] + [
        pl.BlockSpec(be.shape, const2),
        pl.BlockSpec((n, d), const2),
        pl.BlockSpec((d, d), const2),
        pl.BlockSpec((1, d), const2),
    ] + [pl.BlockSpec(t.shape, (lambda r: lambda i, *_: (0,) * r)(t.ndim))
         for t in tail_ops]
    if routed_out:
        out_shape = (jax.ShapeDtypeStruct((n, d), jnp.float32),
                     jax.ShapeDtypeStruct((1, _A + _PAD), jnp.int32),
                     jax.ShapeDtypeStruct((1, _A), jnp.float32))
        out_specs = (pl.BlockSpec((n, d), const2),
                     pl.BlockSpec((1, _A + _PAD), const2),
                     pl.BlockSpec((1, _A), const2))
    else:
        out_shape = jax.ShapeDtypeStruct((n, d), jnp.float32)
        out_specs = pl.BlockSpec((n, d), const2)
    grid_spec = pltpu.PrefetchScalarGridSpec(
        num_scalar_prefetch=2,
        grid=(steps,),
        in_specs=in_specs,
        out_specs=out_specs,
        scratch_shapes=[pltpu.VMEM((n, d), jnp.float32)],
    )
    return pl.pallas_call(kern, grid_spec=grid_spec, out_shape=out_shape)(
        keys, tw, *([we] * _S), be, h, ws, bs, *tail_ops)


def kernel(x, params):
    n = x.shape[0] * x.shape[1]
    xf = x.reshape(n, x.shape[-1]).astype(jnp.float32)
    p0 = params['rl0']
    pf = params['rlf']
    m0 = params['moe0']
    m1 = params['moe1']
    perms = jnp.asarray(_PERMS)
    masks = jnp.asarray(_MASKS)

    h0, keys0, tw0 = pl.pallas_call(
        _stage1_kernel,
        out_shape=(jax.ShapeDtypeStruct((n, _D), jnp.float32),
                   jax.ShapeDtypeStruct((1, _A + _PAD), jnp.int32),
                   jax.ShapeDtypeStruct((1, _A), jnp.float32)),
    )(xf, p0['W1'], _row(p0['b1']), _row(p0['g1']), _row(p0['bb1']),
      p0['W2'], _row(p0['b2']), _row(p0['g2']), _row(p0['bb2']),
      _row(params['ln0_g']), _row(params['ln0_b']),
      m0['Wg'], _row(m0['bg'] + m0['gate_bias']), perms, masks)

    h1, keys1, tw1 = _moe_call(
        _moe_gate_kernel, h0, keys0, tw0, m0['We'], m0['be'],
        m0['Ws'], _row(m0['bs']),
        (m1['Wg'], _row(m1['bg'] + m1['gate_bias']), perms, masks), True)

    y = _moe_call(
        _moe_tail_kernel, h1, keys1, tw1, m1['We'], m1['be'],
        m1['Ws'], _row(m1['bs']),
        (pf['W1'], _row(pf['b1']), _row(pf['g1']), _row(pf['bb1']),
         pf['W2'], _row(pf['b2']), _row(pf['g2']), _row(pf['bb2'])), False)

    return y.reshape(x.shape[:-1] + (y.shape[-1],))
